# baseline (speedup 1.0000x reference)
import jax
import jax.numpy as jnp
from jax.experimental import pallas as pl
from jax.experimental.pallas import tpu as pltpu

_K = 10
_BQ = 512
_BR = 512
_NREF = 100000
_NREF_PAD = 100352
_BIG = 1e30


def _ce(a, b):
    return jnp.minimum(a, b), jnp.maximum(a, b)


def _knn_body(feat_ref, refs_ref, out_ref, flag_ref,
              l_refs, xm2_ref, ybf_ref, y2_ref):
    r = pl.program_id(0)
    q = pl.program_id(1)
    nr = pl.num_programs(0)

    qs = pl.ds(q * _BQ, _BQ)
    lane = jax.lax.broadcasted_iota(jnp.int32, (_BQ, 128), 1)

    @pl.when(r == 0)
    def _init():
        big = jnp.full((_BQ, 128), jnp.inf, jnp.bfloat16)
        for lr in l_refs:
            lr[qs, :] = big
        xm2_ref[qs, :] = (-2.0 * feat_ref[qs, :]).astype(jnp.bfloat16)

    @pl.when(q == 0)
    def _yprep():
        y = refs_ref[...]
        ybf_ref[...] = y.astype(jnp.bfloat16)
        y2_ref[0:1, :] = jnp.sum(y * y, axis=1)[None, :]

    xy = jax.lax.dot_general(
        xm2_ref[qs, :], ybf_ref[...],
        (((1,), (1,)), ((), ())),
        preferred_element_type=jnp.float32)
    e = (y2_ref[0:1, :] + xy).astype(jnp.bfloat16)

    b0, b1 = e[:, 0:128], e[:, 128:256]
    b2, b3 = e[:, 256:384], e[:, 384:512]
    b0, b1 = _ce(b0, b1)
    b2, b3 = _ce(b2, b3)
    b0, b2 = _ce(b0, b2)
    b1, b3 = _ce(b1, b3)
    b1, b2 = _ce(b1, b2)

    m = [lr[qs, :] for lr in l_refs]
    m[2] = jnp.minimum(m[2], b3)
    m[3] = jnp.minimum(m[3], b2)
    m[4] = jnp.minimum(m[4], b1)
    m[5] = jnp.minimum(m[5], b0)
    for i, j in ((1, 4), (2, 5), (4, 5), (0, 2), (1, 2), (2, 3), (3, 4), (4, 5)):
        m[i], m[j] = _ce(m[i], m[j])
    for lr, mi in zip(l_refs, m):
        lr[qs, :] = mi

    @pl.when(r == nr - 1)
    def _fin():
        x = feat_ref[qs, :].astype(jnp.float32)
        x2 = jnp.sum(x * x, axis=1, keepdims=True)
        p = [mi.astype(jnp.float32) for mi in m]
        acc = jnp.zeros((_BQ, 1), jnp.float32)
        for _ in range(_K):
            v = jnp.min(p[0], axis=1, keepdims=True)
            idx = jnp.argmin(p[0], axis=1)[:, None]
            popm = lane == idx
            for lvl in range(5):
                p[lvl] = jnp.where(popm, p[lvl + 1], p[lvl])
            p[5] = jnp.where(popm, _BIG, p[5])
            acc = acc + jnp.sqrt(jnp.maximum(v + x2, 0.0))
        out_ref[...] = -(acc[:, 0] / _K)
        bad = jnp.sum((p[0] >= _BIG).astype(jnp.float32), axis=1)
        flag_ref[...] = bad


def _knn_main(feat, refs_p):
    nq = feat.shape[0] // _BQ
    nr = _NREF_PAD // _BR
    return pl.pallas_call(
        _knn_body,
        grid=(nr, nq),
        in_specs=[
            pl.BlockSpec((feat.shape[0], 128), lambda r, q: (0, 0)),
            pl.BlockSpec((_BR, 128), lambda r, q: (r, 0)),
        ],
        out_specs=[
            pl.BlockSpec((_BQ,), lambda r, q: (q,)),
            pl.BlockSpec((_BQ,), lambda r, q: (q,)),
        ],
        out_shape=[
            jax.ShapeDtypeStruct((feat.shape[0],), jnp.float32),
            jax.ShapeDtypeStruct((feat.shape[0],), jnp.float32),
        ],
        scratch_shapes=[
            [pltpu.VMEM((feat.shape[0], 128), jnp.bfloat16) for _ in range(6)],
            pltpu.VMEM((feat.shape[0], 128), jnp.bfloat16),
            pltpu.VMEM((_BR, 128), jnp.bfloat16),
            pltpu.VMEM((8, _BR), jnp.float32),
        ],
        compiler_params=pltpu.CompilerParams(
            dimension_semantics=("arbitrary", "arbitrary")),
    )(feat, refs_p)


def _exact_body(feat_ref, refs_ref, out_ref, s_ref, xm2_ref, y2_ref):
    r = pl.program_id(0)
    q = pl.program_id(1)
    nr = pl.num_programs(0)

    qs = pl.ds(q * _BQ, _BQ)
    lane = jax.lax.broadcasted_iota(jnp.int32, (_BQ, 128), 1)

    @pl.when(r == 0)
    def _init():
        s_ref[qs, :] = jnp.where(lane < _K, _BIG, -_BIG)
        xm2_ref[qs, :] = (-2.0 * feat_ref[qs, :]).astype(jnp.bfloat16)

    @pl.when(q == 0)
    def _y2():
        y = refs_ref[...]
        y2_ref[0:1, :] = jnp.sum(y * y, axis=1)[None, :]

    xy = jax.lax.dot_general(
        xm2_ref[qs, :], refs_ref[...].astype(jnp.bfloat16),
        (((1,), (1,)), ((), ())),
        preferred_element_type=jnp.float32)
    e = y2_ref[0:1, :] + xy

    c0, c1 = e[:, 0:128], e[:, 128:256]
    c2, c3 = e[:, 256:384], e[:, 384:512]
    c0, c1 = _ce(c0, c1)
    c2, c3 = _ce(c2, c3)
    c0, c2 = _ce(c0, c2)
    c1, c3 = _ce(c1, c3)
    c1, c2 = _ce(c1, c2)

    s = s_ref[qs, :]
    thresh = jnp.max(s, axis=1, keepdims=True)
    mm = jnp.min(c0, axis=1, keepdims=True)
    idx = jnp.argmin(c0, axis=1)[:, None]
    pred = jnp.any(mm < thresh)

    def cond(state):
        return state[0]

    def body(state):
        _, a0, a1, a2, a3, s, thresh, mm, idx = state
        popm = lane == idx
        a0 = jnp.where(popm, a1, a0)
        a1 = jnp.where(popm, a2, a1)
        a2 = jnp.where(popm, a3, a2)
        a3 = jnp.where(popm, _BIG, a3)
        ins = mm < thresh
        imax = jnp.argmax(s, axis=1)[:, None]
        s = jnp.where((lane == imax) & ins, mm, s)
        thresh = jnp.max(s, axis=1, keepdims=True)
        mm = jnp.min(a0, axis=1, keepdims=True)
        idx = jnp.argmin(a0, axis=1)[:, None]
        pred = jnp.any(mm < thresh)
        return pred, a0, a1, a2, a3, s, thresh, mm, idx

    state = (pred, c0, c1, c2, c3, s, thresh, mm, idx)
    state = jax.lax.while_loop(cond, body, state)
    s_ref[qs, :] = state[5]

    @pl.when(r == nr - 1)
    def _fin():
        x = feat_ref[qs, :]
        x2 = jnp.sum(x * x, axis=1, keepdims=True)
        d2 = jnp.maximum(state[5] + x2, 0.0)
        vals = jnp.where(lane < _K, jnp.sqrt(d2), 0.0)
        out_ref[...] = -(jnp.sum(vals, axis=1) / _K)


def _knn_exact(feat, refs_p):
    nq = feat.shape[0] // _BQ
    nr = _NREF_PAD // _BR
    return pl.pallas_call(
        _exact_body,
        grid=(nr, nq),
        in_specs=[
            pl.BlockSpec((feat.shape[0], 128), lambda r, q: (0, 0)),
            pl.BlockSpec((_BR, 128), lambda r, q: (r, 0)),
        ],
        out_specs=pl.BlockSpec((_BQ,), lambda r, q: (q,)),
        out_shape=jax.ShapeDtypeStruct((feat.shape[0],), jnp.float32),
        scratch_shapes=[
            pltpu.VMEM((feat.shape[0], 128), jnp.float32),
            pltpu.VMEM((feat.shape[0], 128), jnp.bfloat16),
            pltpu.VMEM((8, _BR), jnp.float32),
        ],
        compiler_params=pltpu.CompilerParams(
            dimension_semantics=("arbitrary", "arbitrary")),
    )(feat, refs_p)


def kernel(feat, ref_feats):
    refs_p = jnp.pad(ref_feats, ((0, _NREF_PAD - _NREF), (0, 0)),
                     constant_values=1000.0)
    density, flags = _knn_main(feat, refs_p)
    return jax.lax.cond(
        jnp.any(flags > 0.0),
        lambda: _knn_exact(feat, refs_p),
        lambda: density)


# speedup vs baseline: 1.3844x; 1.3844x over previous
import jax
import jax.numpy as jnp
from jax.experimental import pallas as pl
from jax.experimental.pallas import tpu as pltpu

_K = 10
_BQ = 512
_BR = 1024
_BRX = 512
_NREF = 100000
_NREF_PAD = 100352
_BIG = 1e30


def _ce(a, b):
    return jnp.minimum(a, b), jnp.maximum(a, b)


def _knn_body(feat_ref, refs_ref, out_ref, flag_ref,
              l_refs, xm2_ref, ybf_ref, y2_ref):
    r = pl.program_id(0)
    q = pl.program_id(1)
    nr = pl.num_programs(0)

    qs = pl.ds(q * _BQ, _BQ)
    lane = jax.lax.broadcasted_iota(jnp.int32, (_BQ, 128), 1)

    @pl.when(r == 0)
    def _init():
        big = jnp.full((_BQ, 128), _BIG, jnp.float32)
        for lr in l_refs:
            lr[qs, :] = big
        xm2_ref[qs, :] = (-2.0 * feat_ref[qs, :]).astype(jnp.bfloat16)

    @pl.when(q == 0)
    def _yprep():
        y = refs_ref[...]
        ybf_ref[...] = y.astype(jnp.bfloat16)
        y2_ref[0:1, :] = jnp.sum(y * y, axis=1)[None, :]

    xy = jax.lax.dot_general(
        xm2_ref[qs, :], ybf_ref[...],
        (((1,), (1,)), ((), ())),
        preferred_element_type=jnp.float32)
    e = y2_ref[0:1, :] + xy

    a0, a1 = e[:, 0:128], e[:, 128:256]
    a2, a3 = e[:, 256:384], e[:, 384:512]
    b0, b1 = e[:, 512:640], e[:, 640:768]
    b2, b3 = e[:, 768:896], e[:, 896:1024]
    a0, a1 = _ce(a0, a1)
    a2, a3 = _ce(a2, a3)
    a0, a2 = _ce(a0, a2)
    a1, a3 = _ce(a1, a3)
    a1, a2 = _ce(a1, a2)
    b0, b1 = _ce(b0, b1)
    b2, b3 = _ce(b2, b3)
    b0, b2 = _ce(b0, b2)
    b1, b3 = _ce(b1, b3)
    b1, b2 = _ce(b1, b2)

    c = [a0, a1, jnp.minimum(a2, b3), jnp.minimum(a3, b2), b1, b0]
    for i, j in ((0, 4), (2, 4), (1, 5), (3, 5), (2, 3), (0, 1), (4, 5), (1, 4)):
        c[i], c[j] = _ce(c[i], c[j])

    m = [jnp.minimum(lr[qs, :], c[5 - i]) for i, lr in enumerate(l_refs)]
    for i, j in ((1, 5), (0, 4), (2, 4), (3, 5), (2, 3), (4, 5), (0, 1)):
        m[i], m[j] = _ce(m[i], m[j])
    for lr, mi in zip(l_refs, m):
        lr[qs, :] = mi

    @pl.when(r == nr - 1)
    def _fin():
        x = feat_ref[qs, :].astype(jnp.float32)
        x2 = jnp.sum(x * x, axis=1, keepdims=True)
        p = list(m)
        acc = jnp.zeros((_BQ, 1), jnp.float32)
        for _ in range(_K):
            v = jnp.min(p[0], axis=1, keepdims=True)
            idx = jnp.argmin(p[0], axis=1)[:, None]
            popm = lane == idx
            for lvl in range(5):
                p[lvl] = jnp.where(popm, p[lvl + 1], p[lvl])
            p[5] = jnp.where(popm, _BIG, p[5])
            acc = acc + jnp.sqrt(jnp.maximum(v + x2, 0.0))
        out_ref[...] = -(acc[:, 0] / _K)
        bad = jnp.sum((p[0] >= _BIG).astype(jnp.float32), axis=1)
        flag_ref[...] = bad


def _knn_main(feat, refs_p):
    nq = feat.shape[0] // _BQ
    nr = _NREF_PAD // _BR
    return pl.pallas_call(
        _knn_body,
        grid=(nr, nq),
        in_specs=[
            pl.BlockSpec((feat.shape[0], 128), lambda r, q: (0, 0)),
            pl.BlockSpec((_BR, 128), lambda r, q: (r, 0)),
        ],
        out_specs=[
            pl.BlockSpec((_BQ,), lambda r, q: (q,)),
            pl.BlockSpec((_BQ,), lambda r, q: (q,)),
        ],
        out_shape=[
            jax.ShapeDtypeStruct((feat.shape[0],), jnp.float32),
            jax.ShapeDtypeStruct((feat.shape[0],), jnp.float32),
        ],
        scratch_shapes=[
            [pltpu.VMEM((feat.shape[0], 128), jnp.float32) for _ in range(6)],
            pltpu.VMEM((feat.shape[0], 128), jnp.bfloat16),
            pltpu.VMEM((_BR, 128), jnp.bfloat16),
            pltpu.VMEM((8, _BR), jnp.float32),
        ],
        compiler_params=pltpu.CompilerParams(
            dimension_semantics=("arbitrary", "arbitrary")),
    )(feat, refs_p)


def _exact_body(feat_ref, refs_ref, out_ref, s_ref, xm2_ref, y2_ref):
    r = pl.program_id(0)
    q = pl.program_id(1)
    nr = pl.num_programs(0)

    qs = pl.ds(q * _BQ, _BQ)
    lane = jax.lax.broadcasted_iota(jnp.int32, (_BQ, 128), 1)

    @pl.when(r == 0)
    def _init():
        s_ref[qs, :] = jnp.where(lane < _K, _BIG, -_BIG)
        xm2_ref[qs, :] = (-2.0 * feat_ref[qs, :]).astype(jnp.bfloat16)

    @pl.when(q == 0)
    def _y2():
        y = refs_ref[...]
        y2_ref[0:1, :] = jnp.sum(y * y, axis=1)[None, :]

    xy = jax.lax.dot_general(
        xm2_ref[qs, :], refs_ref[...].astype(jnp.bfloat16),
        (((1,), (1,)), ((), ())),
        preferred_element_type=jnp.float32)
    e = y2_ref[0:1, :] + xy

    c0, c1 = e[:, 0:128], e[:, 128:256]
    c2, c3 = e[:, 256:384], e[:, 384:512]
    c0, c1 = _ce(c0, c1)
    c2, c3 = _ce(c2, c3)
    c0, c2 = _ce(c0, c2)
    c1, c3 = _ce(c1, c3)
    c1, c2 = _ce(c1, c2)

    s = s_ref[qs, :]
    thresh = jnp.max(s, axis=1, keepdims=True)
    mm = jnp.min(c0, axis=1, keepdims=True)
    idx = jnp.argmin(c0, axis=1)[:, None]
    pred = jnp.any(mm < thresh)

    def cond(state):
        return state[0]

    def body(state):
        _, a0, a1, a2, a3, s, thresh, mm, idx = state
        popm = lane == idx
        a0 = jnp.where(popm, a1, a0)
        a1 = jnp.where(popm, a2, a1)
        a2 = jnp.where(popm, a3, a2)
        a3 = jnp.where(popm, _BIG, a3)
        ins = mm < thresh
        imax = jnp.argmax(s, axis=1)[:, None]
        s = jnp.where((lane == imax) & ins, mm, s)
        thresh = jnp.max(s, axis=1, keepdims=True)
        mm = jnp.min(a0, axis=1, keepdims=True)
        idx = jnp.argmin(a0, axis=1)[:, None]
        pred = jnp.any(mm < thresh)
        return pred, a0, a1, a2, a3, s, thresh, mm, idx

    state = (pred, c0, c1, c2, c3, s, thresh, mm, idx)
    state = jax.lax.while_loop(cond, body, state)
    s_ref[qs, :] = state[5]

    @pl.when(r == nr - 1)
    def _fin():
        x = feat_ref[qs, :]
        x2 = jnp.sum(x * x, axis=1, keepdims=True)
        d2 = jnp.maximum(state[5] + x2, 0.0)
        vals = jnp.where(lane < _K, jnp.sqrt(d2), 0.0)
        out_ref[...] = -(jnp.sum(vals, axis=1) / _K)


def _knn_exact(feat, refs_p):
    nq = feat.shape[0] // _BQ
    nr = _NREF_PAD // _BRX
    return pl.pallas_call(
        _exact_body,
        grid=(nr, nq),
        in_specs=[
            pl.BlockSpec((feat.shape[0], 128), lambda r, q: (0, 0)),
            pl.BlockSpec((_BRX, 128), lambda r, q: (r, 0)),
        ],
        out_specs=pl.BlockSpec((_BQ,), lambda r, q: (q,)),
        out_shape=jax.ShapeDtypeStruct((feat.shape[0],), jnp.float32),
        scratch_shapes=[
            pltpu.VMEM((feat.shape[0], 128), jnp.float32),
            pltpu.VMEM((feat.shape[0], 128), jnp.bfloat16),
            pltpu.VMEM((8, _BRX), jnp.float32),
        ],
        compiler_params=pltpu.CompilerParams(
            dimension_semantics=("arbitrary", "arbitrary")),
    )(feat, refs_p)


def kernel(feat, ref_feats):
    refs_p = jnp.pad(ref_feats, ((0, _NREF_PAD - _NREF), (0, 0)),
                     constant_values=1000.0)
    density, flags = _knn_main(feat, refs_p)
    return jax.lax.cond(
        jnp.any(flags > 0.0),
        lambda: _knn_exact(feat, refs_p),
        lambda: density)


# speedup vs baseline: 1.5829x; 1.1434x over previous
import jax
import jax.numpy as jnp
from jax.experimental import pallas as pl
from jax.experimental.pallas import tpu as pltpu

_K = 10
_BQ = 512
_BR = 2048
_BRX = 512
_NREF = 100000
_NREF_PAD = 100352
_BIG = 1e30


def _ce(a, b):
    return jnp.minimum(a, b), jnp.maximum(a, b)


def _knn_body(feat_ref, refs_ref, out_ref, flag_ref,
              l_refs, xm2_ref, ybf_ref, y2_ref):
    r = pl.program_id(0)
    q = pl.program_id(1)
    nr = pl.num_programs(0)

    qs = pl.ds(q * _BQ, _BQ)
    lane = jax.lax.broadcasted_iota(jnp.int32, (_BQ, 128), 1)

    @pl.when(r == 0)
    def _init():
        big = jnp.full((_BQ, 128), _BIG, jnp.float32)
        for lr in l_refs:
            lr[qs, :] = big
        xm2_ref[qs, :] = (-2.0 * feat_ref[qs, :]).astype(jnp.bfloat16)

    @pl.when(q == 0)
    def _yprep():
        y = refs_ref[...]
        ybf_ref[...] = y.astype(jnp.bfloat16)
        y2_ref[0:1, :] = jnp.sum(y * y, axis=1)[None, :]

    xy = jax.lax.dot_general(
        xm2_ref[qs, :], ybf_ref[...],
        (((1,), (1,)), ((), ())),
        preferred_element_type=jnp.float32)
    e = y2_ref[0:1, :] + xy

    ch = [e[:, i * 128:(i + 1) * 128] for i in range(16)]

    def _sort4(w, x, yv, z):
        w, x = _ce(w, x)
        yv, z = _ce(yv, z)
        w, yv = _ce(w, yv)
        x, z = _ce(x, z)
        x, yv = _ce(x, yv)
        return w, x, yv, z

    def _low6_44(A, B):
        v = [A[0], A[1], jnp.minimum(A[2], B[3]), jnp.minimum(A[3], B[2]),
             B[1], B[0]]
        for i, j in ((0, 4), (2, 4), (1, 5), (3, 5), (2, 3), (0, 1), (4, 5),
                     (1, 4)):
            v[i], v[j] = _ce(v[i], v[j])
        return v

    def _low6_66(L, C):
        v = [jnp.minimum(L[i], C[5 - i]) for i in range(6)]
        for i, j in ((1, 5), (0, 4), (2, 4), (3, 5), (2, 3), (4, 5), (0, 1)):
            v[i], v[j] = _ce(v[i], v[j])
        return v

    qd = [_sort4(*ch[4 * i:4 * i + 4]) for i in range(4)]
    cand = _low6_66(_low6_44(qd[0], qd[1]), _low6_44(qd[2], qd[3]))
    m = _low6_66([lr[qs, :] for lr in l_refs], cand)
    for lr, mi in zip(l_refs, m):
        lr[qs, :] = mi

    @pl.when(r == nr - 1)
    def _fin():
        x = feat_ref[qs, :].astype(jnp.float32)
        x2 = jnp.sum(x * x, axis=1, keepdims=True)
        p = list(m)
        acc = jnp.zeros((_BQ, 1), jnp.float32)
        for _ in range(_K):
            v = jnp.min(p[0], axis=1, keepdims=True)
            idx = jnp.argmin(p[0], axis=1)[:, None]
            popm = lane == idx
            for lvl in range(5):
                p[lvl] = jnp.where(popm, p[lvl + 1], p[lvl])
            p[5] = jnp.where(popm, _BIG, p[5])
            acc = acc + jnp.sqrt(jnp.maximum(v + x2, 0.0))
        out_ref[...] = -(acc[:, 0] / _K)
        bad = jnp.sum((p[0] >= _BIG).astype(jnp.float32), axis=1)
        flag_ref[...] = bad


def _knn_main(feat, refs_p):
    nq = feat.shape[0] // _BQ
    nr = _NREF_PAD // _BR
    return pl.pallas_call(
        _knn_body,
        grid=(nr, nq),
        in_specs=[
            pl.BlockSpec((feat.shape[0], 128), lambda r, q: (0, 0)),
            pl.BlockSpec((_BR, 128), lambda r, q: (r, 0)),
        ],
        out_specs=[
            pl.BlockSpec((_BQ,), lambda r, q: (q,)),
            pl.BlockSpec((_BQ,), lambda r, q: (q,)),
        ],
        out_shape=[
            jax.ShapeDtypeStruct((feat.shape[0],), jnp.float32),
            jax.ShapeDtypeStruct((feat.shape[0],), jnp.float32),
        ],
        scratch_shapes=[
            [pltpu.VMEM((feat.shape[0], 128), jnp.float32) for _ in range(6)],
            pltpu.VMEM((feat.shape[0], 128), jnp.bfloat16),
            pltpu.VMEM((_BR, 128), jnp.bfloat16),
            pltpu.VMEM((8, _BR), jnp.float32),
        ],
        compiler_params=pltpu.CompilerParams(
            dimension_semantics=("arbitrary", "arbitrary")),
    )(feat, refs_p)


def _exact_body(feat_ref, refs_ref, out_ref, s_ref, xm2_ref, y2_ref):
    r = pl.program_id(0)
    q = pl.program_id(1)
    nr = pl.num_programs(0)

    qs = pl.ds(q * _BQ, _BQ)
    lane = jax.lax.broadcasted_iota(jnp.int32, (_BQ, 128), 1)

    @pl.when(r == 0)
    def _init():
        s_ref[qs, :] = jnp.where(lane < _K, _BIG, -_BIG)
        xm2_ref[qs, :] = (-2.0 * feat_ref[qs, :]).astype(jnp.bfloat16)

    @pl.when(q == 0)
    def _y2():
        y = refs_ref[...]
        y2_ref[0:1, :] = jnp.sum(y * y, axis=1)[None, :]

    xy = jax.lax.dot_general(
        xm2_ref[qs, :], refs_ref[...].astype(jnp.bfloat16),
        (((1,), (1,)), ((), ())),
        preferred_element_type=jnp.float32)
    e = y2_ref[0:1, :] + xy

    c0, c1 = e[:, 0:128], e[:, 128:256]
    c2, c3 = e[:, 256:384], e[:, 384:512]
    c0, c1 = _ce(c0, c1)
    c2, c3 = _ce(c2, c3)
    c0, c2 = _ce(c0, c2)
    c1, c3 = _ce(c1, c3)
    c1, c2 = _ce(c1, c2)

    s = s_ref[qs, :]
    thresh = jnp.max(s, axis=1, keepdims=True)
    mm = jnp.min(c0, axis=1, keepdims=True)
    idx = jnp.argmin(c0, axis=1)[:, None]
    pred = jnp.any(mm < thresh)

    def cond(state):
        return state[0]

    def body(state):
        _, a0, a1, a2, a3, s, thresh, mm, idx = state
        popm = lane == idx
        a0 = jnp.where(popm, a1, a0)
        a1 = jnp.where(popm, a2, a1)
        a2 = jnp.where(popm, a3, a2)
        a3 = jnp.where(popm, _BIG, a3)
        ins = mm < thresh
        imax = jnp.argmax(s, axis=1)[:, None]
        s = jnp.where((lane == imax) & ins, mm, s)
        thresh = jnp.max(s, axis=1, keepdims=True)
        mm = jnp.min(a0, axis=1, keepdims=True)
        idx = jnp.argmin(a0, axis=1)[:, None]
        pred = jnp.any(mm < thresh)
        return pred, a0, a1, a2, a3, s, thresh, mm, idx

    state = (pred, c0, c1, c2, c3, s, thresh, mm, idx)
    state = jax.lax.while_loop(cond, body, state)
    s_ref[qs, :] = state[5]

    @pl.when(r == nr - 1)
    def _fin():
        x = feat_ref[qs, :]
        x2 = jnp.sum(x * x, axis=1, keepdims=True)
        d2 = jnp.maximum(state[5] + x2, 0.0)
        vals = jnp.where(lane < _K, jnp.sqrt(d2), 0.0)
        out_ref[...] = -(jnp.sum(vals, axis=1) / _K)


def _knn_exact(feat, refs_p):
    nq = feat.shape[0] // _BQ
    nr = _NREF_PAD // _BRX
    return pl.pallas_call(
        _exact_body,
        grid=(nr, nq),
        in_specs=[
            pl.BlockSpec((feat.shape[0], 128), lambda r, q: (0, 0)),
            pl.BlockSpec((_BRX, 128), lambda r, q: (r, 0)),
        ],
        out_specs=pl.BlockSpec((_BQ,), lambda r, q: (q,)),
        out_shape=jax.ShapeDtypeStruct((feat.shape[0],), jnp.float32),
        scratch_shapes=[
            pltpu.VMEM((feat.shape[0], 128), jnp.float32),
            pltpu.VMEM((feat.shape[0], 128), jnp.bfloat16),
            pltpu.VMEM((8, _BRX), jnp.float32),
        ],
        compiler_params=pltpu.CompilerParams(
            dimension_semantics=("arbitrary", "arbitrary")),
    )(feat, refs_p)


def kernel(feat, ref_feats):
    refs_p = jnp.pad(ref_feats, ((0, _NREF_PAD - _NREF), (0, 0)),
                     constant_values=1000.0)
    density, flags = _knn_main(feat, refs_p)
    return jax.lax.cond(
        jnp.any(flags > 0.0),
        lambda: _knn_exact(feat, refs_p),
        lambda: density)
